# baseline (speedup 1.0000x reference)
import functools

import jax
import jax.numpy as jnp
from jax import lax
from jax.experimental import pallas as pl
from jax.experimental.pallas import tpu as pltpu
from jax.experimental.pallas import tpu_sc as plsc

_N = 10000
_E = 320000
_D = 128
_NW = 32
_BLK = 128
_NB = 80
_EPW = _NB * _BLK
_EPAD = _NW * _EPW
_NPAD = 10240
_ELAST = _E - (_NW - 1) * _EPW
_IGRP = 40
_RPT = _NPAD // 16
_BN = 1000

_mesh = plsc.VectorSubcoreMesh(core_axis_name="c", subcore_axis_name="s")
_SC_PARAMS = pltpu.CompilerParams(needs_layout_passes=False)


@functools.partial(
    pl.kernel,
    out_type=(
        jax.ShapeDtypeStruct((_NW * _NB, _BLK), jnp.int32),
        jax.ShapeDtypeStruct((_NW * _NB, _BLK), jnp.int32),
        jax.ShapeDtypeStruct((2, _NPAD), jnp.float32),
    ),
    mesh=_mesh,
    compiler_params=_SC_PARAMS,
    scratch_types=[
        pltpu.VMEM((_N,), jnp.int32),
        pltpu.VMEM((_EPW,), jnp.int32),
        pltpu.VMEM((_EPW,), jnp.int32),
        pltpu.VMEM((_NB, _BLK), jnp.int32),
        pltpu.VMEM((_NB, _BLK), jnp.int32),
        pltpu.VMEM((_NB, _BLK), jnp.float32),
        pltpu.VMEM((_RPT,), jnp.float32),
        pltpu.VMEM_SHARED((_NPAD,), jnp.float32),
        pltpu.SemaphoreType.DMA,
    ],
)
def _sc_pre(edge_hbm, lev_hbm, gidx_hbm, rowadj_hbm, deg_hbm,
            lev_v, row_v, col_v, gidx_v, rowadj_v, w_v, zero_v, deg_s, sem):
    cid = lax.axis_index("c")
    sid = lax.axis_index("s")
    wid = sid * 2 + cid
    base = wid * _EPW

    pltpu.sync_copy(lev_hbm, lev_v)

    @pl.when(wid < _NW - 1)
    def _():
        pltpu.sync_copy(edge_hbm.at[pl.ds(base, _EPW)], row_v)
        pltpu.sync_copy(edge_hbm.at[pl.ds(_E + base, _EPW)], col_v)

    @pl.when(wid == _NW - 1)
    def _():
        pltpu.sync_copy(edge_hbm.at[pl.ds(base, _ELAST)],
                        row_v.at[pl.ds(0, _ELAST)])
        pltpu.sync_copy(edge_hbm.at[pl.ds(_E + base, _ELAST)],
                        col_v.at[pl.ds(0, _ELAST)])

    def zb(k, c):
        zero_v[pl.ds(k * 16, 16)] = jnp.zeros((16,), jnp.float32)
        return c
    lax.fori_loop(0, _RPT // 16, zb, 0)
    pltpu.sync_copy(zero_v, deg_s.at[pl.ds(sid * _RPT, _RPT)])

    def body(j, c):
        for u in range(8):
            off = j * _BLK + u * 16
            eid = base + off + lax.iota(jnp.int32, 16)
            valid = eid < _E
            r = jnp.where(valid, row_v[pl.ds(off, 16)], 0)
            cc = jnp.where(valid, col_v[pl.ds(off, 16)], 0)
            lr = plsc.load_gather(lev_v, [r])
            lc = plsc.load_gather(lev_v, [cc])
            d = jnp.abs(lr - lc)
            spread = jnp.bitwise_and(eid, 7)
            g = jnp.where(valid, d * _N + cc, spread)
            ra = jnp.where(valid, r, _N + spread)
            w = jnp.where(valid, 1.0 / (1.0 + d.astype(jnp.float32)),
                          jnp.zeros((16,), jnp.float32))
            co = u * 16
            gidx_v[j, pl.ds(co, 16)] = g
            rowadj_v[j, pl.ds(co, 16)] = ra
            w_v[j, pl.ds(co, 16)] = w
        return c
    lax.fori_loop(0, _NB, body, 0)

    pltpu.sync_copy(gidx_v, gidx_hbm.at[pl.ds(wid * _NB, _NB)])
    pltpu.sync_copy(rowadj_v, rowadj_hbm.at[pl.ds(wid * _NB, _NB)])

    plsc.subcore_barrier()
    descs = []
    for j in range(_NB):
        descs.append(pltpu.async_copy(
            w_v.at[j], deg_s.at[rowadj_v.at[j]], sem, add=True))
    for dsc in descs:
        dsc.wait()
    plsc.subcore_barrier()
    pltpu.sync_copy(deg_s.at[pl.ds(sid * _RPT, _RPT)],
                    deg_hbm.at[cid, pl.ds(sid * _RPT, _RPT)])


@functools.partial(
    pl.kernel,
    out_type=jax.ShapeDtypeStruct((2, _NPAD, _D), jnp.float32),
    mesh=_mesh,
    compiler_params=_SC_PARAMS,
    scratch_types=[
        pltpu.VMEM((2 * _IGRP, _BLK), jnp.int32),
        pltpu.VMEM((_BLK, _D), jnp.float32),
        pltpu.VMEM((_BLK, _D), jnp.float32),
        pltpu.VMEM_SHARED((_NPAD, _D), jnp.float32),
        pltpu.SemaphoreType.DMA,
        pltpu.SemaphoreType.DMA,
    ],
)
def _sc_edge(h3_hbm, gidx_hbm, rowadj_hbm, out_hbm,
             idx_v, buf0, buf1, agg_s, sg0, sg1):
    cid = lax.axis_index("c")
    sid = lax.axis_index("s")
    wid = sid * 2 + cid
    bb = wid * _NB

    def zb(r, c):
        for u in range(8):
            buf0[r, pl.ds(u * 16, 16)] = jnp.zeros((16,), jnp.float32)
        return c
    lax.fori_loop(0, _BLK, zb, 0)
    for t in range(_RPT // _BLK):
        pltpu.sync_copy(buf0, agg_s.at[pl.ds(sid * _RPT + t * _BLK, _BLK)])
    plsc.subcore_barrier()

    def fire(b, buf, sem):
        pltpu.async_copy(h3_hbm.at[idx_v.at[b]], buf, sem)

    def gwait(b, buf, sem):
        pltpu.make_async_copy(h3_hbm.at[idx_v.at[b]], buf, sem).wait()

    def scat(b, buf):
        pltpu.sync_copy(buf, agg_s.at[idx_v.at[_IGRP + b]], add=True)

    def group(g, c):
        gb = bb + g * _IGRP
        pltpu.sync_copy(gidx_hbm.at[pl.ds(gb, _IGRP)],
                        idx_v.at[pl.ds(0, _IGRP)])
        pltpu.sync_copy(rowadj_hbm.at[pl.ds(gb, _IGRP)],
                        idx_v.at[pl.ds(_IGRP, _IGRP)])
        fire(0, buf0, sg0)

        def body(i, c2):
            b0 = 2 * i
            fire(b0 + 1, buf1, sg1)
            gwait(b0, buf0, sg0)
            scat(b0, buf0)
            fire(b0 + 2, buf0, sg0)
            gwait(b0 + 1, buf1, sg1)
            scat(b0 + 1, buf1)
            return c2
        lax.fori_loop(0, _IGRP // 2 - 1, body, 0)
        bl = _IGRP - 2
        fire(bl + 1, buf1, sg1)
        gwait(bl, buf0, sg0)
        scat(bl, buf0)
        gwait(bl + 1, buf1, sg1)
        scat(bl + 1, buf1)
        return c
    lax.fori_loop(0, _NB // _IGRP, group, 0)

    plsc.subcore_barrier()
    pltpu.sync_copy(agg_s.at[pl.ds(sid * _RPT, _RPT)],
                    out_hbm.at[cid, pl.ds(sid * _RPT, _RPT)])


def _onehot(lev_block, n):
    return (lev_block == lax.broadcasted_iota(jnp.int32, (n, 8), 1)
            ).astype(jnp.float32)


def _dense_h(feats, lev_ref, w_ref, lev_enc_ref, b_ref):
    hp = lax.Precision.HIGHEST
    cdim1 = (((1,), (1,)), ((), ()))
    w = w_ref[...]
    m = lax.dot_general(lev_enc_ref[...], w[:, _D:], cdim1, precision=hp)
    m = jnp.pad(m, ((0, 5), (0, 0)))
    h = lax.dot_general(feats, w[:, :_D], cdim1, precision=hp)
    h = h + jnp.dot(_onehot(lev_ref[...], feats.shape[0]), m, precision=hp)
    return h + b_ref[...]


def _emit_h3(h, out_ref):
    out_ref[0] = h
    out_ref[1] = jnp.float32(0.5) * h
    out_ref[2] = jnp.float32(1.0 / 3.0) * h


def _combine(h, part_ref, deg_ref):
    agg = part_ref[0] + part_ref[1]
    deg = deg_ref[0] + deg_ref[1]
    agg = agg / (deg + 1e-8)
    o = 0.5 * (h + agg)
    nrm = jnp.sqrt(jnp.sum(o * o, axis=1, keepdims=True))
    return o / (nrm + 1e-8)


def _dense0_body(x_ref, lev_ref, w_ref, levpad_ref, b_ref, out_ref):
    _emit_h3(_dense_h(x_ref[...], lev_ref, w_ref, levpad_ref, b_ref),
             out_ref)


def _mid_body(h3_ref, part_ref, deg_ref, lev_ref, w_ref, levpad_ref,
              b_ref, out_ref):
    o = _combine(h3_ref[0], part_ref, deg_ref)
    o = jnp.maximum(o, 0.0)
    nrm = jnp.sqrt(jnp.sum(o * o, axis=1, keepdims=True))
    o = o / (nrm + 1e-8)
    _emit_h3(_dense_h(o, lev_ref, w_ref, levpad_ref, b_ref), out_ref)


def _final_body(h3_ref, part_ref, deg_ref, out_ref):
    out_ref[...] = _combine(h3_ref[0], part_ref, deg_ref)[:, :_D - 1]


def _rep(shape):
    nd = len(shape)
    return pl.BlockSpec(shape, lambda i: (0,) * nd)


_GRID = (_N // _BN,)
_s_h3blk = pl.BlockSpec((1, _BN, _D), lambda i: (0, i, 0))
_s_part = pl.BlockSpec((2, _BN, _D), lambda i: (0, i, 0))
_s_deg = pl.BlockSpec((2, _BN, 1), lambda i: (0, i, 0))
_s_lev = pl.BlockSpec((_BN, 1), lambda i: (i, 0))
_s_h3out = pl.BlockSpec((3, _BN, _D), lambda i: (0, i, 0))
_DENSE_W_SPECS = [_rep((_D, _D + 8)), _rep((3, 8)), _rep((1, _D))]


def _dense0(x, lev2d, w, levpad, b2d):
    return pl.pallas_call(
        _dense0_body,
        grid=_GRID,
        in_specs=[pl.BlockSpec((_BN, _D), lambda i: (i, 0)), _s_lev,
                  *_DENSE_W_SPECS],
        out_specs=_s_h3out,
        out_shape=jax.ShapeDtypeStruct((3, _N, _D), jnp.float32),
    )(x, lev2d, w, levpad, b2d)


def _mid(h3_0, part, deg3, lev2d, w, levpad, b2d):
    return pl.pallas_call(
        _mid_body,
        grid=_GRID,
        in_specs=[_s_h3blk, _s_part, _s_deg, _s_lev, *_DENSE_W_SPECS],
        out_specs=_s_h3out,
        out_shape=jax.ShapeDtypeStruct((3, _N, _D), jnp.float32),
    )(h3_0, part, deg3, lev2d, w, levpad, b2d)


def _final(h3_1, part, deg3):
    return pl.pallas_call(
        _final_body,
        grid=_GRID,
        in_specs=[_s_h3blk, _s_part, _s_deg],
        out_specs=pl.BlockSpec((_BN, _D - 1), lambda i: (i, 0)),
        out_shape=jax.ShapeDtypeStruct((_N, _D - 1), jnp.float32),
    )(h3_1, part, deg3)


def kernel(x, edge_index, node_levels, W0, b0, lev0, W1, b1, lev1):
    gidx, rowadj, deg2 = _sc_pre(edge_index.reshape(2 * _E), node_levels)
    deg3 = deg2[:, :, None]
    lev2d = node_levels[:, None]

    h3_0 = _dense0(x, lev2d, W0, lev0, b0[None])
    p0 = _sc_edge(h3_0.reshape(3 * _N, _D), gidx, rowadj)

    h3_1 = _mid(h3_0, p0, deg3, lev2d, W1, lev1, b1[None])
    p1 = _sc_edge(h3_1.reshape(3 * _N, _D), gidx, rowadj)

    return _final(h3_1, p1, deg3)
